# baseline (speedup 1.0000x reference)
import functools

import jax
import jax.numpy as jnp
from jax import lax
from jax.experimental import pallas as pl
from jax.experimental.pallas import tpu as pltpu
from jax.experimental.pallas import tpu_sc as plsc

_VOCAB = 100000
_EMBED = 16
_BATCH = 1024
_HIST = 200

_NC, _NS = 2, 16
_NW = _NC * _NS
_CHUNK = 100
_CPR = _HIST // _CHUNK
_GROWS = 4
_GCHUNKS = _GROWS * _CPR


def _make_pool(rows_w):
    ngroups = rows_w // _GROWS
    batch = rows_w * _NW

    def body(x_hbm, table_hbm, out_hbm, idx_v, buf_a, buf_b, pooled_v,
             sem_a, sem_b):
        wid = lax.axis_index("s") * _NC + lax.axis_index("c")
        pltpu.sync_copy(x_hbm.at[wid, pl.ds(0, _GCHUNKS)],
                        idx_v.at[pl.ds(0, _GCHUNKS)])

        def fire(g, buf, sem):
            for k in range(_GCHUNKS):
                c = g * _GCHUNKS + k
                pltpu.async_copy(
                    table_hbm.at[idx_v.at[c]],
                    buf.at[pl.ds(k * _CHUNK, _CHUNK)],
                    sem,
                )

        def drain(buf, sem):
            for k in range(_GCHUNKS):
                pltpu.make_async_copy(
                    table_hbm.at[idx_v.at[k]],
                    buf.at[pl.ds(k * _CHUNK, _CHUNK)],
                    sem,
                ).wait()

        def acc_group(g, buf):
            for r in range(_GROWS):
                base = r * _HIST
                accs = [jnp.zeros((_EMBED,), jnp.float32) for _ in range(4)]
                for j in range(0, _HIST, 4):
                    for t in range(4):
                        accs[t] = accs[t] + buf[base + j + t]
                acc = (accs[0] + accs[1]) + (accs[2] + accs[3])
                pooled_v[g * _GROWS + r] = acc * (1.0 / _HIST)

        fire(0, buf_a, sem_a)
        pltpu.sync_copy(
            x_hbm.at[wid, pl.ds(_GCHUNKS, rows_w * _CPR - _GCHUNKS)],
            idx_v.at[pl.ds(_GCHUNKS, rows_w * _CPR - _GCHUNKS)],
        )

        def pair_body(p, carry):
            g0 = 2 * p
            fire(g0 + 1, buf_b, sem_b)
            drain(buf_a, sem_a)
            acc_group(g0, buf_a)

            @pl.when(p < ngroups // 2 - 1)
            def _():
                fire(g0 + 2, buf_a, sem_a)

            drain(buf_b, sem_b)
            acc_group(g0 + 1, buf_b)
            return carry

        lax.fori_loop(0, ngroups // 2, pair_body, 0)
        pltpu.sync_copy(pooled_v, out_hbm.at[pl.ds(wid * rows_w, rows_w)])

    return pl.kernel(
        body,
        out_type=jax.ShapeDtypeStruct((batch, _EMBED), jnp.float32),
        mesh=plsc.VectorSubcoreMesh(core_axis_name="c", subcore_axis_name="s"),
        scratch_types=[
            pltpu.VMEM((rows_w * _CPR, _CHUNK), jnp.int32),
            pltpu.VMEM((_GCHUNKS * _CHUNK, _EMBED), jnp.float32),
            pltpu.VMEM((_GCHUNKS * _CHUNK, _EMBED), jnp.float32),
            pltpu.VMEM((rows_w, _EMBED), jnp.float32),
            pltpu.SemaphoreType.DMA,
            pltpu.SemaphoreType.DMA,
        ],
        compiler_params=pltpu.CompilerParams(use_tc_tiling_on_sc=False),
    )


_ROWS_W = _BATCH // _NW
_pool = _make_pool(_ROWS_W)

_TV = 4096


def _mm_body(p_ref, w_ref, b_ref, o_ref):
    o_ref[...] = (
        jnp.dot(p_ref[...], w_ref[...], preferred_element_type=jnp.float32)
        + b_ref[...]
    )


def _project(pooled, W, b2d):
    return pl.pallas_call(
        _mm_body,
        grid=(pl.cdiv(_VOCAB, _TV),),
        in_specs=[
            pl.BlockSpec((_BATCH, _EMBED), lambda v: (0, 0)),
            pl.BlockSpec((_EMBED, _TV), lambda v: (0, v)),
            pl.BlockSpec((1, _TV), lambda v: (0, v)),
        ],
        out_specs=pl.BlockSpec((_BATCH, _TV), lambda v: (0, v)),
        out_shape=jax.ShapeDtypeStruct((_BATCH, _VOCAB), jnp.float32),
    )(pooled, W, b2d)


def kernel(x, table, W, b):
    x_r = x.reshape(_NW, _ROWS_W * _CPR, _CHUNK)
    pooled = _pool(x_r, table)
    return _project(pooled, W, b.reshape(1, _VOCAB))


# speedup vs baseline: 1.0141x; 1.0141x over previous
import functools

import jax
import jax.numpy as jnp
from jax import lax
from jax.experimental import pallas as pl
from jax.experimental.pallas import tpu as pltpu
from jax.experimental.pallas import tpu_sc as plsc

_VOCAB = 100000
_EMBED = 16
_BATCH = 1024
_HIST = 200

_NC, _NS = 2, 16
_NW = _NC * _NS
_CHUNK = 100
_CPR = _HIST // _CHUNK
_GROWS = 4
_GCHUNKS = _GROWS * _CPR


def _make_pool(rows_w):
    ngroups = rows_w // _GROWS
    batch = rows_w * _NW

    def body(x_hbm, table_hbm, out_hbm, idx_v, buf_a, buf_b, pooled_v,
             sem_a, sem_b):
        wid = lax.axis_index("s") * _NC + lax.axis_index("c")
        pltpu.sync_copy(x_hbm.at[wid, pl.ds(0, _GCHUNKS)],
                        idx_v.at[pl.ds(0, _GCHUNKS)])

        def fire(g, buf, sem):
            for k in range(_GCHUNKS):
                c = g * _GCHUNKS + k
                pltpu.async_copy(
                    table_hbm.at[idx_v.at[c]],
                    buf.at[pl.ds(k * _CHUNK, _CHUNK)],
                    sem,
                )

        def drain(buf, sem):
            for k in range(_GCHUNKS):
                pltpu.make_async_copy(
                    table_hbm.at[idx_v.at[k]],
                    buf.at[pl.ds(k * _CHUNK, _CHUNK)],
                    sem,
                ).wait()

        def acc_group(g, buf):
            for r in range(_GROWS):
                base = r * _HIST

                def add8(j, acc, base=base, buf=buf):
                    o = base + j * 8
                    return acc + (
                        ((buf[o] + buf[o + 1]) + (buf[o + 2] + buf[o + 3]))
                        + ((buf[o + 4] + buf[o + 5]) + (buf[o + 6] + buf[o + 7]))
                    )

                acc = lax.fori_loop(
                    0, _HIST // 8, add8, jnp.zeros((_EMBED,), jnp.float32),
                    unroll=2,
                )
                pooled_v[g * _GROWS + r] = acc * (1.0 / _HIST)

        fire(0, buf_a, sem_a)
        pltpu.sync_copy(
            x_hbm.at[wid, pl.ds(_GCHUNKS, rows_w * _CPR - _GCHUNKS)],
            idx_v.at[pl.ds(_GCHUNKS, rows_w * _CPR - _GCHUNKS)],
        )

        def pair_body(p, carry):
            g0 = 2 * p
            fire(g0 + 1, buf_b, sem_b)
            drain(buf_a, sem_a)
            acc_group(g0, buf_a)

            @pl.when(p < ngroups // 2 - 1)
            def _():
                fire(g0 + 2, buf_a, sem_a)

            drain(buf_b, sem_b)
            acc_group(g0 + 1, buf_b)
            return carry

        lax.fori_loop(0, ngroups // 2, pair_body, 0)
        pltpu.sync_copy(pooled_v, out_hbm.at[pl.ds(wid * rows_w, rows_w)])

    return pl.kernel(
        body,
        out_type=jax.ShapeDtypeStruct((batch, _EMBED), jnp.float32),
        mesh=plsc.VectorSubcoreMesh(core_axis_name="c", subcore_axis_name="s"),
        scratch_types=[
            pltpu.VMEM((rows_w * _CPR, _CHUNK), jnp.int32),
            pltpu.VMEM((_GCHUNKS * _CHUNK, _EMBED), jnp.float32),
            pltpu.VMEM((_GCHUNKS * _CHUNK, _EMBED), jnp.float32),
            pltpu.VMEM((rows_w, _EMBED), jnp.float32),
            pltpu.SemaphoreType.DMA,
            pltpu.SemaphoreType.DMA,
        ],
        compiler_params=pltpu.CompilerParams(use_tc_tiling_on_sc=False),
    )


_ROWS_W = _BATCH // _NW
_pool = _make_pool(_ROWS_W)

_TV = 4096


def _mm_body(p_ref, w_ref, b_ref, o_ref):
    o_ref[...] = (
        jnp.dot(p_ref[...], w_ref[...], preferred_element_type=jnp.float32)
        + b_ref[...]
    )


def _project(pooled, W, b2d):
    return pl.pallas_call(
        _mm_body,
        grid=(pl.cdiv(_VOCAB, _TV),),
        in_specs=[
            pl.BlockSpec((_BATCH, _EMBED), lambda v: (0, 0)),
            pl.BlockSpec((_EMBED, _TV), lambda v: (0, v)),
            pl.BlockSpec((1, _TV), lambda v: (0, v)),
        ],
        out_specs=pl.BlockSpec((_BATCH, _TV), lambda v: (0, v)),
        out_shape=jax.ShapeDtypeStruct((_BATCH, _VOCAB), jnp.float32),
    )(pooled, W, b2d)


def kernel(x, table, W, b):
    x_r = x.reshape(_NW, _ROWS_W * _CPR, _CHUNK)
    pooled = _pool(x_r, table)
    return _project(pooled, W, b.reshape(1, _VOCAB))


# speedup vs baseline: 1.0169x; 1.0027x over previous
import functools

import jax
import jax.numpy as jnp
from jax import lax
from jax.experimental import pallas as pl
from jax.experimental.pallas import tpu as pltpu
from jax.experimental.pallas import tpu_sc as plsc

_VOCAB = 100000
_EMBED = 16
_BATCH = 1024
_HIST = 200

_NC, _NS = 2, 16
_NW = _NC * _NS
_CHUNK = 100
_CPR = _HIST // _CHUNK
_GROWS = 4
_GCHUNKS = _GROWS * _CPR


def _make_pool(rows_w):
    ngroups = rows_w // _GROWS
    batch = rows_w * _NW

    def body(x_hbm, table_hbm, out_hbm, idx_v, buf_a, buf_b, pooled_v,
             sem_a, sem_b):
        wid = lax.axis_index("s") * _NC + lax.axis_index("c")
        pltpu.sync_copy(x_hbm.at[wid, pl.ds(0, _GCHUNKS)],
                        idx_v.at[pl.ds(0, _GCHUNKS)])

        def fire(g, buf, sem):
            for k in range(_GCHUNKS):
                c = g * _GCHUNKS + k
                pltpu.async_copy(
                    table_hbm.at[idx_v.at[c]],
                    buf.at[pl.ds(k * _CHUNK, _CHUNK)],
                    sem,
                )

        def drain(buf, sem):
            pltpu.make_async_copy(
                table_hbm.at[pl.ds(0, _GCHUNKS * _CHUNK)],
                buf,
                sem,
            ).wait()

        def acc_group(g, buf):
            for r in range(_GROWS):
                base = r * _HIST

                def add8(j, acc, base=base, buf=buf):
                    o = base + j * 8
                    return acc + (
                        ((buf[o] + buf[o + 1]) + (buf[o + 2] + buf[o + 3]))
                        + ((buf[o + 4] + buf[o + 5]) + (buf[o + 6] + buf[o + 7]))
                    )

                acc = lax.fori_loop(
                    0, _HIST // 8, add8, jnp.zeros((_EMBED,), jnp.float32),
                    unroll=5,
                )
                pooled_v[g * _GROWS + r] = acc * (1.0 / _HIST)

        fire(0, buf_a, sem_a)
        pltpu.sync_copy(
            x_hbm.at[wid, pl.ds(_GCHUNKS, rows_w * _CPR - _GCHUNKS)],
            idx_v.at[pl.ds(_GCHUNKS, rows_w * _CPR - _GCHUNKS)],
        )

        def pair_body(p, carry):
            g0 = 2 * p
            fire(g0 + 1, buf_b, sem_b)
            drain(buf_a, sem_a)
            acc_group(g0, buf_a)

            @pl.when(p < ngroups // 2 - 1)
            def _():
                fire(g0 + 2, buf_a, sem_a)

            drain(buf_b, sem_b)
            acc_group(g0 + 1, buf_b)
            return carry

        lax.fori_loop(0, ngroups // 2, pair_body, 0)
        pltpu.sync_copy(pooled_v, out_hbm.at[pl.ds(wid * rows_w, rows_w)])

    return pl.kernel(
        body,
        out_type=jax.ShapeDtypeStruct((batch, _EMBED), jnp.float32),
        mesh=plsc.VectorSubcoreMesh(core_axis_name="c", subcore_axis_name="s"),
        scratch_types=[
            pltpu.VMEM((rows_w * _CPR, _CHUNK), jnp.int32),
            pltpu.VMEM((_GCHUNKS * _CHUNK, _EMBED), jnp.float32),
            pltpu.VMEM((_GCHUNKS * _CHUNK, _EMBED), jnp.float32),
            pltpu.VMEM((rows_w, _EMBED), jnp.float32),
            pltpu.SemaphoreType.DMA,
            pltpu.SemaphoreType.DMA,
        ],
        compiler_params=pltpu.CompilerParams(use_tc_tiling_on_sc=False),
    )


_ROWS_W = _BATCH // _NW
_pool = _make_pool(_ROWS_W)

_TV = 4096


def _mm_body(p_ref, w_ref, b_ref, o_ref):
    o_ref[...] = (
        jnp.dot(p_ref[...], w_ref[...], preferred_element_type=jnp.float32)
        + b_ref[...]
    )


def _project(pooled, W, b2d):
    return pl.pallas_call(
        _mm_body,
        grid=(pl.cdiv(_VOCAB, _TV),),
        in_specs=[
            pl.BlockSpec((_BATCH, _EMBED), lambda v: (0, 0)),
            pl.BlockSpec((_EMBED, _TV), lambda v: (0, v)),
            pl.BlockSpec((1, _TV), lambda v: (0, v)),
        ],
        out_specs=pl.BlockSpec((_BATCH, _TV), lambda v: (0, v)),
        out_shape=jax.ShapeDtypeStruct((_BATCH, _VOCAB), jnp.float32),
    )(pooled, W, b2d)


def kernel(x, table, W, b):
    x_r = x.reshape(_NW, _ROWS_W * _CPR, _CHUNK)
    pooled = _pool(x_r, table)
    return _project(pooled, W, b.reshape(1, _VOCAB))
